# baseline (speedup 1.0000x reference)
import functools
import math

import jax
import jax.numpy as jnp
from jax import lax
from jax.experimental import pallas as pl
from jax.experimental.pallas import tpu as pltpu
from jax.experimental.pallas import tpu_sc as plsc

NN = 10000
EE = 160000
DD = 256
HH = 4
CC = DD // HH

NC = 2
NS = 16
LANES = 16


def _q_body(x_ref, wq_ref, q_ref, xb_ref):
    xv = x_ref[...]
    q_ref[...] = jnp.dot(xv, wq_ref[...],
                         preferred_element_type=jnp.float32
                         ).astype(jnp.bfloat16)
    xb_ref[...] = xv.astype(jnp.bfloat16)


def _tc_query(x, Wq):
    BN = 1000
    return pl.pallas_call(
        _q_body,
        grid=(NN // BN,),
        in_specs=[pl.BlockSpec((BN, DD), lambda i: (i, 0)),
                  pl.BlockSpec((DD, DD), lambda i: (0, 0))],
        out_specs=[pl.BlockSpec((BN, DD), lambda i: (i, 0)),
                   pl.BlockSpec((BN, DD), lambda i: (i, 0))],
        out_shape=[jax.ShapeDtypeStruct((NN, DD), jnp.bfloat16),
                   jax.ShapeDtypeStruct((NN, DD), jnp.bfloat16)],
    )(x, Wq)


def _sc_gather(x, query, dst, src):
    mesh = plsc.VectorSubcoreMesh(core_axis_name="c", subcore_axis_name="s")
    DH = DD // 2
    EW = EE // (NC * NS)
    CH = 128
    NF = EW // CH
    TL = EW - NF * CH

    NB = 3
    NFI = NF // NB

    @functools.partial(
        pl.kernel,
        out_type=(jax.ShapeDtypeStruct((EE, DH), jnp.int32),
                  jax.ShapeDtypeStruct((EE, DH), jnp.int32)),
        mesh=mesh,
        scratch_types=[pltpu.VMEM((EW,), jnp.int32),
                       pltpu.VMEM((EW,), jnp.int32),
                       pltpu.VMEM((CH, DH), jnp.int32),
                       pltpu.VMEM((CH, DH), jnp.int32),
                       pltpu.VMEM((CH, DH), jnp.int32),
                       pltpu.SemaphoreType.DMA,
                       pltpu.SemaphoreType.DMA,
                       pltpu.SemaphoreType.DMA,
                       pltpu.SemaphoreType.DMA,
                       pltpu.SemaphoreType.DMA,
                       pltpu.SemaphoreType.DMA],
    )
    def k(x_hbm, q_hbm, dst_hbm, src_hbm, xd_hbm, qs_hbm,
          idxd, idxs, b0, b1, b2, g0, g1, g2, w0, w1, w2):
        wid = lax.axis_index("s") * NC + lax.axis_index("c")
        base = wid * EW
        bufs = (b0, b1, b2)
        gsem = (g0, g1, g2)
        wsem = (w0, w1, w2)

        pltpu.sync_copy(dst_hbm.at[pl.ds(base, EW)], idxd)
        pltpu.sync_copy(src_hbm.at[pl.ds(base, EW)], idxs)

        def run(tab_hbm, idxall, out_hbm):
            def gather(c, j):
                pltpu.async_copy(tab_hbm.at[idxall.at[pl.ds(c * CH, CH)]],
                                 bufs[j], gsem[j])

            for j in range(NB):
                gather(j, j)

            def body(i, carry):
                for j in range(NB):
                    c = i * NB + j
                    pltpu.make_async_copy(
                        tab_hbm.at[idxall.at[pl.ds(c * CH, CH)]],
                        bufs[j], gsem[j]).wait()
                    pltpu.async_copy(bufs[j],
                                     out_hbm.at[pl.ds(base + c * CH, CH)],
                                     wsem[j])

                    @pl.when(i < NFI - 1)
                    def _(j=j, c=c):
                        pltpu.make_async_copy(
                            bufs[j],
                            out_hbm.at[pl.ds(base + c * CH, CH)],
                            wsem[j]).wait()
                        gather(c + NB, j)
                return carry

            lax.fori_loop(0, NFI, body, 0)
            for j in range(NB):
                pltpu.make_async_copy(
                    bufs[j],
                    out_hbm.at[pl.ds(base + (NF - NB + j) * CH, CH)],
                    wsem[j]).wait()
            pltpu.async_copy(tab_hbm.at[idxall.at[pl.ds(NF * CH, TL)]],
                             bufs[0].at[pl.ds(0, TL), :], gsem[0])
            pltpu.make_async_copy(tab_hbm.at[idxall.at[pl.ds(NF * CH, TL)]],
                                  bufs[0].at[pl.ds(0, TL), :], gsem[0]).wait()
            pltpu.sync_copy(bufs[0].at[pl.ds(0, TL), :],
                            out_hbm.at[pl.ds(base + NF * CH, TL)])

        run(x_hbm, idxd, xd_hbm)
        run(q_hbm, idxs, qs_hbm)

    return k(x, query, dst, src)


def _att_body(ea_ref, xd_ref, qs_ref, wk_ref, wv_ref, we_ref, be_ref,
              sh_ref, v_ref, eo_ref, att_ref):
    z = (ea_ref[...] * xd_ref[...].astype(jnp.float32)).astype(jnp.bfloat16)
    kk = jnp.dot(z, wk_ref[...], preferred_element_type=jnp.float32)
    v_ref[...] = jnp.dot(z, wv_ref[...], preferred_element_type=jnp.float32
                         ).astype(jnp.bfloat16)
    eo_ref[...] = (jnp.dot(z, we_ref[...], preferred_element_type=jnp.float32)
                   + be_ref[...])
    p = (qs_ref[...].astype(jnp.float32) * kk).astype(jnp.bfloat16)
    att = jnp.dot(p, sh_ref[...],
                  preferred_element_type=jnp.float32)
    att_ref[...] = att.T


def _tc_proj(edge_attr, xd, qs, Wk, Wv, We, be):
    BE = 640
    Wk = Wk.astype(jnp.bfloat16)
    Wv = Wv.astype(jnp.bfloat16)
    We = We.astype(jnp.bfloat16)
    shead = (jnp.repeat(jnp.eye(HH, dtype=jnp.float32), CC, axis=0)
             * (1.0 / math.sqrt(CC))).astype(jnp.bfloat16)
    be2 = be.reshape(1, DD)
    return pl.pallas_call(
        _att_body,
        grid=(EE // BE,),
        in_specs=[pl.BlockSpec((BE, DD), lambda i: (i, 0)),
                  pl.BlockSpec((BE, DD), lambda i: (i, 0)),
                  pl.BlockSpec((BE, DD), lambda i: (i, 0)),
                  pl.BlockSpec((DD, DD), lambda i: (0, 0)),
                  pl.BlockSpec((DD, DD), lambda i: (0, 0)),
                  pl.BlockSpec((DD, DD), lambda i: (0, 0)),
                  pl.BlockSpec((1, DD), lambda i: (0, 0)),
                  pl.BlockSpec((DD, HH), lambda i: (0, 0))],
        out_specs=[pl.BlockSpec((BE, DD), lambda i: (i, 0)),
                   pl.BlockSpec((BE, DD), lambda i: (i, 0)),
                   pl.BlockSpec((HH, BE), lambda i: (0, i))],
        out_shape=[jax.ShapeDtypeStruct((EE, DD), jnp.bfloat16),
                   jax.ShapeDtypeStruct((EE, DD), jnp.float32),
                   jax.ShapeDtypeStruct((HH, EE), jnp.float32)],
    )(edge_attr, xd, qs, Wk, Wv, We, be2, shead)


def _sc_softmax(att_flat, src):
    mesh = plsc.VectorSubcoreMesh(core_axis_name="c", subcore_axis_name="s",
                                  num_cores=1)
    ET = EE // NS
    CH = 128
    NF = ET // CH
    TL = ET - NF * CH
    ZT = (HH * NN) // NS // 8 * 8
    ZR = HH * NN - ZT * NS

    @functools.partial(
        pl.kernel,
        out_type=jax.ShapeDtypeStruct((HH * EE,), jnp.float32),
        mesh=mesh,
        scratch_types=[
            pltpu.VMEM((HH * ET,), jnp.float32),
            pltpu.VMEM((HH * NN,), jnp.float32),
            pltpu.VMEM((ET,), jnp.int32),
            pltpu.VMEM((CH,), jnp.int32),
            pltpu.VMEM((CH,), jnp.int32),
            pltpu.VMEM((CH,), jnp.int32),
            pltpu.VMEM((CH,), jnp.int32),
            pltpu.VMEM((CH,), jnp.int32),
            pltpu.VMEM((CH,), jnp.int32),
            pltpu.VMEM((TL,), jnp.int32),
            pltpu.VMEM_SHARED((HH * NN,), jnp.float32),
            pltpu.VMEM_SHARED((HH * NN,), jnp.float32),
            pltpu.SemaphoreType.DMA,
            pltpu.SemaphoreType.DMA,
            pltpu.SemaphoreType.DMA,
            pltpu.SemaphoreType.DMA,
            pltpu.SemaphoreType.DMA,
            pltpu.SemaphoreType.DMA,
        ],
        compiler_params=pltpu.CompilerParams(needs_layout_passes=False),
    )
    def k(att_hbm, src_hbm, coeff_hbm, av, tab, srcv,
          r0, r1, r2, r3, r4, r5, idxt, mtab_s, den_s,
          s0, s1, s2, s3, s4, s5):
        rbuf = (r0, r1, r2, r3, r4, r5)
        rsem = (s0, s1, s2, s3, s4, s5)
        t = lax.axis_index("s")
        ebase = t * ET

        zv = jnp.zeros((LANES,), jnp.float32)

        def zbody(g, carry):
            av[pl.ds(g * LANES, LANES)] = zv
            return carry
        lax.fori_loop(0, ZT // LANES, zbody, 0)
        pltpu.sync_copy(av.at[pl.ds(0, ZT)], den_s.at[pl.ds(t * ZT, ZT)])

        @pl.when(t == NS - 1)
        def _():
            pltpu.sync_copy(av.at[pl.ds(0, ZR)],
                            den_s.at[pl.ds(NS * ZT, ZR)])

        pltpu.sync_copy(src_hbm.at[pl.ds(ebase, ET)], srcv)
        for h in range(HH):
            pltpu.sync_copy(att_hbm.at[pl.ds(h * EE + ebase, ET)],
                            av.at[pl.ds(h * ET, ET)])

        def build_idx(off, h, idxr, n):
            for j in range(n // LANES):
                s16 = srcv[pl.ds(off + j * LANES, LANES)]
                idxr[pl.ds(j * LANES, LANES)] = s16 + h * NN

        RD = 6
        RI = NF // RD

        def scat_pass(h, dst_s, add):
            def body(i, carry):
                for j in range(RD):
                    c = i * RD + j
                    off = h * ET + c * CH

                    @pl.when(i > 0)
                    def _(j=j, off=off):
                        pltpu.make_async_copy(av.at[pl.ds(off, CH)],
                                              dst_s.at[rbuf[j]],
                                              rsem[j]).wait()
                    build_idx(c * CH, h, rbuf[j], CH)
                    pltpu.async_copy(av.at[pl.ds(off, CH)],
                                     dst_s.at[rbuf[j]], rsem[j], add=add)
                return carry
            lax.fori_loop(0, RI, body, 0)
            for j in range(RD):
                off = h * ET + ((RI - 1) * RD + j) * CH
                pltpu.make_async_copy(av.at[pl.ds(off, CH)],
                                      dst_s.at[rbuf[j]], rsem[j]).wait()
            build_idx(NF * CH, h, idxt, TL)
            pltpu.sync_copy(av.at[pl.ds(h * ET + NF * CH, TL)],
                            dst_s.at[idxt], add=add)

        for h in range(HH):
            scat_pass(h, mtab_s, False)

        plsc.subcore_barrier()
        pltpu.sync_copy(mtab_s, tab)

        for h in range(HH):
            def exbody(g, carry, h=h):
                s16 = srcv[pl.ds(g * LANES, LANES)]
                m = plsc.load_gather(tab, [s16 + h * NN])
                a = av[pl.ds(h * ET + g * LANES, LANES)]
                av[pl.ds(h * ET + g * LANES, LANES)] = jnp.exp(a - m)
                return carry
            lax.fori_loop(0, ET // LANES, exbody, 0)

        for h in range(HH):
            scat_pass(h, den_s, True)

        plsc.subcore_barrier()
        pltpu.sync_copy(den_s, tab)

        for h in range(HH):
            def cbody(g, carry, h=h):
                s16 = srcv[pl.ds(g * LANES, LANES)]
                d = plsc.load_gather(tab, [s16 + h * NN])
                e = av[pl.ds(h * ET + g * LANES, LANES)]
                av[pl.ds(h * ET + g * LANES, LANES)] = e / (d + 1e-16)
                return carry
            lax.fori_loop(0, ET // LANES, cbody, 0)
        for h in range(HH):
            pltpu.sync_copy(av.at[pl.ds(h * ET, ET)],
                            coeff_hbm.at[pl.ds(h * EE + ebase, ET)])

    return k(att_flat, src)


def _w_body(coeff_ref, v_ref, ex_ref, w_ref):
    scale = jnp.dot(coeff_ref[...], ex_ref[...],
                    preferred_element_type=jnp.float32)
    w = v_ref[...].astype(jnp.float32) * scale
    w_ref[:, 0, :] = w[:, :DD // 2]
    w_ref[:, 1, :] = w[:, DD // 2:]


def _tc_scale(coeff, v):
    BE = 2000
    expand = jnp.repeat(jnp.eye(HH, dtype=jnp.float32), CC, axis=1)
    return pl.pallas_call(
        _w_body,
        grid=(EE // BE,),
        in_specs=[pl.BlockSpec((BE, HH), lambda i: (i, 0)),
                  pl.BlockSpec((BE, DD), lambda i: (i, 0)),
                  pl.BlockSpec((HH, DD), lambda i: (0, 0))],
        out_specs=pl.BlockSpec((BE, 2, DD // 2), lambda i: (i, 0, 0)),
        out_shape=jax.ShapeDtypeStruct((EE, 2, DD // 2), jnp.float32),
    )(coeff, v, expand)


def _sc_scatter_out(w2, src, zeros2d):
    mesh = plsc.VectorSubcoreMesh(core_axis_name="c", subcore_axis_name="s")
    ET = EE // NS
    CH = 128
    NF = ET // CH
    TL = ET - NF * CH
    RT = 624
    HW = DD // 2

    @functools.partial(
        pl.kernel,
        out_type=jax.ShapeDtypeStruct((NN, 2, HW), jnp.float32),
        mesh=mesh,
        scratch_types=[
            pltpu.VMEM((CH, HW), jnp.float32),
            pltpu.VMEM((CH, HW), jnp.float32),
            pltpu.VMEM((TL, HW), jnp.float32),
            pltpu.VMEM((CH,), jnp.int32),
            pltpu.VMEM((CH,), jnp.int32),
            pltpu.VMEM((TL,), jnp.int32),
            pltpu.VMEM_SHARED((NN, HW), jnp.float32),
            pltpu.SemaphoreType.DMA,
            pltpu.SemaphoreType.DMA,
            pltpu.SemaphoreType.DMA,
            pltpu.SemaphoreType.DMA,
        ],
    )
    def k(w_hbm, src_hbm, z_hbm, out_hbm, wb0, wb1, wbuft,
          ix0, ix1, idxt, acc_s, g0, g1, a0, a1):
        c = lax.axis_index("c")
        t = lax.axis_index("s")
        wbufs = (wb0, wb1)
        sidx = (ix0, ix1)
        gsem = (g0, g1)
        asem = (a0, a1)

        pltpu.sync_copy(z_hbm.at[pl.ds(0, RT), :],
                        acc_s.at[pl.ds(t * RT, RT), :])

        @pl.when(t == NS - 1)
        def _():
            pltpu.sync_copy(z_hbm.at[pl.ds(0, 16), :],
                            acc_s.at[pl.ds(NS * RT, 16), :])

        plsc.subcore_barrier()

        base = t * ET
        RD = 2
        RI = NF // RD

        def loadw(cc, j):
            pltpu.sync_copy(src_hbm.at[pl.ds(base + cc * CH, CH)], sidx[j])
            pltpu.async_copy(w_hbm.at[pl.ds(base + cc * CH, CH), c, :],
                             wbufs[j], gsem[j])

        for j in range(RD):
            loadw(j, j)

        def scat(i, carry):
            for j in range(RD):
                cc = i * RD + j
                pltpu.make_async_copy(
                    w_hbm.at[pl.ds(base + cc * CH, CH), c, :],
                    wbufs[j], gsem[j]).wait()
                pltpu.async_copy(wbufs[j], acc_s.at[sidx[j]], asem[j],
                                 add=True)

                @pl.when(i < RI - 1)
                def _(j=j, cc=cc):
                    pltpu.make_async_copy(wbufs[j], acc_s.at[sidx[j]],
                                          asem[j]).wait()
                    loadw(cc + RD, j)
            return carry
        lax.fori_loop(0, RI, scat, 0)
        for j in range(RD):
            pltpu.make_async_copy(wbufs[j], acc_s.at[sidx[j]],
                                  asem[j]).wait()
        offt = base + NF * CH
        pltpu.sync_copy(src_hbm.at[pl.ds(offt, TL)], idxt)
        pltpu.sync_copy(w_hbm.at[pl.ds(offt, TL), c, :], wbuft)
        pltpu.sync_copy(wbuft, acc_s.at[idxt], add=True)

        plsc.subcore_barrier()

        pltpu.sync_copy(acc_s.at[pl.ds(t * RT, RT), :],
                        out_hbm.at[pl.ds(t * RT, RT), c, :])

        @pl.when(t == NS - 1)
        def _():
            pltpu.sync_copy(acc_s.at[pl.ds(NS * RT, 16), :],
                            out_hbm.at[pl.ds(NS * RT, 16), c, :])

    return k(w2, src, zeros2d)


def kernel(x, edge_index, edge_attr, Wq, Wk, Wv, We, be):
    src = edge_index[0]
    dst = edge_index[1]
    query, xb = _tc_query(x, Wq)
    qi = lax.bitcast_convert_type(query.reshape(NN, DD // 2, 2), jnp.int32)
    xi = lax.bitcast_convert_type(xb.reshape(NN, DD // 2, 2), jnp.int32)
    xdi, qsi = _sc_gather(xi, qi, dst, src)
    xd = lax.bitcast_convert_type(xdi, jnp.bfloat16).reshape(EE, DD)
    qs = lax.bitcast_convert_type(qsi, jnp.bfloat16).reshape(EE, DD)
    v, eout, att_t = _tc_proj(edge_attr, xd, qs, Wk, Wv, We, be)
    coeff_flat = _sc_softmax(att_t.reshape(HH * EE), src)
    w2 = _tc_scale(coeff_flat.reshape(HH, EE).T, v)
    z2 = jnp.zeros((624, DD // 2), jnp.float32)
    out2 = _sc_scatter_out(w2, src, z2)
    return out2.reshape(NN, DD), eout


# speedup vs baseline: 2.6234x; 2.6234x over previous
import functools
import math

import jax
import jax.numpy as jnp
from jax import lax
from jax.experimental import pallas as pl
from jax.experimental.pallas import tpu as pltpu
from jax.experimental.pallas import tpu_sc as plsc

NN = 10000
EE = 160000
DD = 256
HH = 4
CC = DD // HH

NC = 2
NS = 16
LANES = 16


def _pack2(v):
    b = lax.bitcast_convert_type(v, jnp.int32)
    r = b + 0x7FFF + ((b >> 16) & 1)
    lo = (r[:, :DD // 2] >> 16) & 0xFFFF
    hi = (r[:, DD // 2:] >> 16) << 16
    return hi | lo


def _unpack2(w):
    lo = lax.bitcast_convert_type(w << 16, jnp.float32)
    hi = lax.bitcast_convert_type((w >> 16) << 16, jnp.float32)
    return jnp.concatenate([lo, hi], axis=1)


def _q_body(x_ref, wq_ref, q_ref, xb_ref):
    xv = x_ref[...]
    q = jnp.dot(xv, wq_ref[...], preferred_element_type=jnp.float32)
    q_ref[...] = _pack2(q)
    xb_ref[...] = _pack2(xv)


def _tc_query(x, Wq):
    BN = 1000
    return pl.pallas_call(
        _q_body,
        grid=(NN // BN,),
        in_specs=[pl.BlockSpec((BN, DD), lambda i: (i, 0)),
                  pl.BlockSpec((DD, DD), lambda i: (0, 0))],
        out_specs=[pl.BlockSpec((BN, DD // 2), lambda i: (i, 0)),
                   pl.BlockSpec((BN, DD // 2), lambda i: (i, 0))],
        out_shape=[jax.ShapeDtypeStruct((NN, DD // 2), jnp.int32),
                   jax.ShapeDtypeStruct((NN, DD // 2), jnp.int32)],
    )(x, Wq)


def _sc_gather(x, query, dst, src):
    mesh = plsc.VectorSubcoreMesh(core_axis_name="c", subcore_axis_name="s")
    DH = DD // 2
    EW = EE // (NC * NS)
    CH = 128
    NF = EW // CH
    TL = EW - NF * CH

    NB = 3
    NFI = NF // NB

    @functools.partial(
        pl.kernel,
        out_type=(jax.ShapeDtypeStruct((EE, DH), jnp.int32),
                  jax.ShapeDtypeStruct((EE, DH), jnp.int32)),
        mesh=mesh,
        scratch_types=[pltpu.VMEM((EW,), jnp.int32),
                       pltpu.VMEM((EW,), jnp.int32),
                       pltpu.VMEM((CH, DH), jnp.int32),
                       pltpu.VMEM((CH, DH), jnp.int32),
                       pltpu.VMEM((CH, DH), jnp.int32),
                       pltpu.SemaphoreType.DMA,
                       pltpu.SemaphoreType.DMA,
                       pltpu.SemaphoreType.DMA,
                       pltpu.SemaphoreType.DMA,
                       pltpu.SemaphoreType.DMA,
                       pltpu.SemaphoreType.DMA],
    )
    def k(x_hbm, q_hbm, dst_hbm, src_hbm, xd_hbm, qs_hbm,
          idxd, idxs, b0, b1, b2, g0, g1, g2, w0, w1, w2):
        wid = lax.axis_index("s") * NC + lax.axis_index("c")
        base = wid * EW
        bufs = (b0, b1, b2)
        gsem = (g0, g1, g2)
        wsem = (w0, w1, w2)

        pltpu.sync_copy(dst_hbm.at[pl.ds(base, EW)], idxd)
        pltpu.sync_copy(src_hbm.at[pl.ds(base, EW)], idxs)

        def run(tab_hbm, idxall, out_hbm):
            def gather(c, j):
                pltpu.async_copy(tab_hbm.at[idxall.at[pl.ds(c * CH, CH)]],
                                 bufs[j], gsem[j])

            for j in range(NB):
                gather(j, j)

            def body(i, carry):
                for j in range(NB):
                    c = i * NB + j
                    pltpu.make_async_copy(
                        tab_hbm.at[idxall.at[pl.ds(c * CH, CH)]],
                        bufs[j], gsem[j]).wait()
                    pltpu.async_copy(bufs[j],
                                     out_hbm.at[pl.ds(base + c * CH, CH)],
                                     wsem[j])

                    @pl.when(i < NFI - 1)
                    def _(j=j, c=c):
                        pltpu.make_async_copy(
                            bufs[j],
                            out_hbm.at[pl.ds(base + c * CH, CH)],
                            wsem[j]).wait()
                        gather(c + NB, j)
                return carry

            lax.fori_loop(0, NFI, body, 0)
            for j in range(NB):
                pltpu.make_async_copy(
                    bufs[j],
                    out_hbm.at[pl.ds(base + (NF - NB + j) * CH, CH)],
                    wsem[j]).wait()
            pltpu.async_copy(tab_hbm.at[idxall.at[pl.ds(NF * CH, TL)]],
                             bufs[0].at[pl.ds(0, TL), :], gsem[0])
            pltpu.make_async_copy(tab_hbm.at[idxall.at[pl.ds(NF * CH, TL)]],
                                  bufs[0].at[pl.ds(0, TL), :], gsem[0]).wait()
            pltpu.sync_copy(bufs[0].at[pl.ds(0, TL), :],
                            out_hbm.at[pl.ds(base + NF * CH, TL)])

        run(x_hbm, idxd, xd_hbm)
        run(q_hbm, idxs, qs_hbm)

    return k(x, query, dst, src)


def _att_body(ea_ref, xd_ref, qs_ref, wk_ref, wv_ref, we_ref, be_ref,
              sh_ref, v_ref, eo_ref, att_ref):
    z = (ea_ref[...] * _unpack2(xd_ref[...])).astype(jnp.bfloat16)
    kk = jnp.dot(z, wk_ref[...], preferred_element_type=jnp.float32)
    v_ref[...] = jnp.dot(z, wv_ref[...], preferred_element_type=jnp.float32
                         ).astype(jnp.bfloat16)
    eo_ref[...] = (jnp.dot(z, we_ref[...], preferred_element_type=jnp.float32)
                   + be_ref[...])
    p = (_unpack2(qs_ref[...]) * kk).astype(jnp.bfloat16)
    att = jnp.dot(p, sh_ref[...],
                  preferred_element_type=jnp.float32)
    att_ref[...] = att.T


def _tc_proj(edge_attr, xd, qs, Wk, Wv, We, be):
    BE = 640
    Wk = Wk.astype(jnp.bfloat16)
    Wv = Wv.astype(jnp.bfloat16)
    We = We.astype(jnp.bfloat16)
    shead = (jnp.repeat(jnp.eye(HH, dtype=jnp.float32), CC, axis=0)
             * (1.0 / math.sqrt(CC))).astype(jnp.bfloat16)
    be2 = be.reshape(1, DD)
    return pl.pallas_call(
        _att_body,
        grid=(EE // BE,),
        in_specs=[pl.BlockSpec((BE, DD), lambda i: (i, 0)),
                  pl.BlockSpec((BE, DD // 2), lambda i: (i, 0)),
                  pl.BlockSpec((BE, DD // 2), lambda i: (i, 0)),
                  pl.BlockSpec((DD, DD), lambda i: (0, 0)),
                  pl.BlockSpec((DD, DD), lambda i: (0, 0)),
                  pl.BlockSpec((DD, DD), lambda i: (0, 0)),
                  pl.BlockSpec((1, DD), lambda i: (0, 0)),
                  pl.BlockSpec((DD, HH), lambda i: (0, 0))],
        out_specs=[pl.BlockSpec((BE, DD), lambda i: (i, 0)),
                   pl.BlockSpec((BE, DD), lambda i: (i, 0)),
                   pl.BlockSpec((HH, BE), lambda i: (0, i))],
        out_shape=[jax.ShapeDtypeStruct((EE, DD), jnp.bfloat16),
                   jax.ShapeDtypeStruct((EE, DD), jnp.float32),
                   jax.ShapeDtypeStruct((HH, EE), jnp.float32)],
    )(edge_attr, xd, qs, Wk, Wv, We, be2, shead)


def _sc_softmax(att_flat, src):
    mesh = plsc.VectorSubcoreMesh(core_axis_name="c", subcore_axis_name="s",
                                  num_cores=1)
    ET = EE // NS
    CH = 128
    NF = ET // CH
    TL = ET - NF * CH
    ZT = (HH * NN) // NS // 8 * 8
    ZR = HH * NN - ZT * NS

    @functools.partial(
        pl.kernel,
        out_type=jax.ShapeDtypeStruct((HH * EE,), jnp.float32),
        mesh=mesh,
        scratch_types=[
            pltpu.VMEM((HH * ET,), jnp.float32),
            pltpu.VMEM((HH * NN,), jnp.float32),
            pltpu.VMEM((ET,), jnp.int32),
            pltpu.VMEM((CH,), jnp.int32),
            pltpu.VMEM((CH,), jnp.int32),
            pltpu.VMEM((CH,), jnp.int32),
            pltpu.VMEM((CH,), jnp.int32),
            pltpu.VMEM((CH,), jnp.int32),
            pltpu.VMEM((CH,), jnp.int32),
            pltpu.VMEM((TL,), jnp.int32),
            pltpu.VMEM_SHARED((HH * NN,), jnp.float32),
            pltpu.VMEM_SHARED((HH * NN,), jnp.float32),
            pltpu.SemaphoreType.DMA,
            pltpu.SemaphoreType.DMA,
            pltpu.SemaphoreType.DMA,
            pltpu.SemaphoreType.DMA,
            pltpu.SemaphoreType.DMA,
            pltpu.SemaphoreType.DMA,
        ],
        compiler_params=pltpu.CompilerParams(needs_layout_passes=False),
    )
    def k(att_hbm, src_hbm, coeff_hbm, av, tab, srcv,
          r0, r1, r2, r3, r4, r5, idxt, mtab_s, den_s,
          s0, s1, s2, s3, s4, s5):
        rbuf = (r0, r1, r2, r3, r4, r5)
        rsem = (s0, s1, s2, s3, s4, s5)
        t = lax.axis_index("s")
        ebase = t * ET

        zv = jnp.zeros((LANES,), jnp.float32)

        def zbody(g, carry):
            av[pl.ds(g * LANES, LANES)] = zv
            return carry
        lax.fori_loop(0, ZT // LANES, zbody, 0)
        pltpu.sync_copy(av.at[pl.ds(0, ZT)], den_s.at[pl.ds(t * ZT, ZT)])

        @pl.when(t == NS - 1)
        def _():
            pltpu.sync_copy(av.at[pl.ds(0, ZR)],
                            den_s.at[pl.ds(NS * ZT, ZR)])

        pltpu.sync_copy(src_hbm.at[pl.ds(ebase, ET)], srcv)
        for h in range(HH):
            pltpu.sync_copy(att_hbm.at[pl.ds(h * EE + ebase, ET)],
                            av.at[pl.ds(h * ET, ET)])

        def build_idx(off, h, idxr, n):
            for j in range(n // LANES):
                s16 = srcv[pl.ds(off + j * LANES, LANES)]
                idxr[pl.ds(j * LANES, LANES)] = s16 + h * NN

        RD = 6
        RI = NF // RD

        def scat_pass(h, dst_s, add):
            def body(i, carry):
                for j in range(RD):
                    c = i * RD + j
                    off = h * ET + c * CH

                    @pl.when(i > 0)
                    def _(j=j, off=off):
                        pltpu.make_async_copy(av.at[pl.ds(off, CH)],
                                              dst_s.at[rbuf[j]],
                                              rsem[j]).wait()
                    build_idx(c * CH, h, rbuf[j], CH)
                    pltpu.async_copy(av.at[pl.ds(off, CH)],
                                     dst_s.at[rbuf[j]], rsem[j], add=add)
                return carry
            lax.fori_loop(0, RI, body, 0)
            for j in range(RD):
                off = h * ET + ((RI - 1) * RD + j) * CH
                pltpu.make_async_copy(av.at[pl.ds(off, CH)],
                                      dst_s.at[rbuf[j]], rsem[j]).wait()
            build_idx(NF * CH, h, idxt, TL)
            pltpu.sync_copy(av.at[pl.ds(h * ET + NF * CH, TL)],
                            dst_s.at[idxt], add=add)

        for h in range(HH):
            scat_pass(h, mtab_s, False)

        plsc.subcore_barrier()
        pltpu.sync_copy(mtab_s, tab)

        for h in range(HH):
            def exbody(g, carry, h=h):
                s16 = srcv[pl.ds(g * LANES, LANES)]
                m = plsc.load_gather(tab, [s16 + h * NN])
                a = av[pl.ds(h * ET + g * LANES, LANES)]
                av[pl.ds(h * ET + g * LANES, LANES)] = jnp.exp(a - m)
                return carry
            lax.fori_loop(0, ET // LANES, exbody, 0)

        for h in range(HH):
            scat_pass(h, den_s, True)

        plsc.subcore_barrier()
        pltpu.sync_copy(den_s, tab)

        for h in range(HH):
            def cbody(g, carry, h=h):
                s16 = srcv[pl.ds(g * LANES, LANES)]
                d = plsc.load_gather(tab, [s16 + h * NN])
                e = av[pl.ds(h * ET + g * LANES, LANES)]
                av[pl.ds(h * ET + g * LANES, LANES)] = e / (d + 1e-16)
                return carry
            lax.fori_loop(0, ET // LANES, cbody, 0)
        for h in range(HH):
            pltpu.sync_copy(av.at[pl.ds(h * ET, ET)],
                            coeff_hbm.at[pl.ds(h * EE + ebase, ET)])

    return k(att_flat, src)


def _w_body(coeff_ref, v_ref, ex_ref, w_ref):
    scale = jnp.dot(coeff_ref[...], ex_ref[...],
                    preferred_element_type=jnp.float32)
    w = v_ref[...].astype(jnp.float32) * scale
    w_ref[:, 0, :] = w[:, :DD // 2]
    w_ref[:, 1, :] = w[:, DD // 2:]


def _tc_scale(coeff, v):
    BE = 2000
    expand = jnp.repeat(jnp.eye(HH, dtype=jnp.float32), CC, axis=1)
    return pl.pallas_call(
        _w_body,
        grid=(EE // BE,),
        in_specs=[pl.BlockSpec((BE, HH), lambda i: (i, 0)),
                  pl.BlockSpec((BE, DD), lambda i: (i, 0)),
                  pl.BlockSpec((HH, DD), lambda i: (0, 0))],
        out_specs=pl.BlockSpec((BE, 2, DD // 2), lambda i: (i, 0, 0)),
        out_shape=jax.ShapeDtypeStruct((EE, 2, DD // 2), jnp.float32),
    )(coeff, v, expand)


def _sc_scatter_out(w2, src, zeros2d):
    mesh = plsc.VectorSubcoreMesh(core_axis_name="c", subcore_axis_name="s")
    ET = EE // NS
    CH = 128
    NF = ET // CH
    TL = ET - NF * CH
    RT = 624
    HW = DD // 2

    @functools.partial(
        pl.kernel,
        out_type=jax.ShapeDtypeStruct((NN, 2, HW), jnp.float32),
        mesh=mesh,
        scratch_types=[
            pltpu.VMEM((CH, HW), jnp.float32),
            pltpu.VMEM((CH, HW), jnp.float32),
            pltpu.VMEM((TL, HW), jnp.float32),
            pltpu.VMEM((CH,), jnp.int32),
            pltpu.VMEM((CH,), jnp.int32),
            pltpu.VMEM((TL,), jnp.int32),
            pltpu.VMEM_SHARED((NN, HW), jnp.float32),
            pltpu.SemaphoreType.DMA,
            pltpu.SemaphoreType.DMA,
            pltpu.SemaphoreType.DMA,
            pltpu.SemaphoreType.DMA,
        ],
    )
    def k(w_hbm, src_hbm, z_hbm, out_hbm, wb0, wb1, wbuft,
          ix0, ix1, idxt, acc_s, g0, g1, a0, a1):
        c = lax.axis_index("c")
        t = lax.axis_index("s")
        wbufs = (wb0, wb1)
        sidx = (ix0, ix1)
        gsem = (g0, g1)
        asem = (a0, a1)

        pltpu.sync_copy(z_hbm.at[pl.ds(0, RT), :],
                        acc_s.at[pl.ds(t * RT, RT), :])

        @pl.when(t == NS - 1)
        def _():
            pltpu.sync_copy(z_hbm.at[pl.ds(0, 16), :],
                            acc_s.at[pl.ds(NS * RT, 16), :])

        plsc.subcore_barrier()

        base = t * ET
        RD = 2
        RI = NF // RD

        def loadw(cc, j):
            pltpu.sync_copy(src_hbm.at[pl.ds(base + cc * CH, CH)], sidx[j])
            pltpu.async_copy(w_hbm.at[pl.ds(base + cc * CH, CH), c, :],
                             wbufs[j], gsem[j])

        for j in range(RD):
            loadw(j, j)

        def scat(i, carry):
            for j in range(RD):
                cc = i * RD + j
                pltpu.make_async_copy(
                    w_hbm.at[pl.ds(base + cc * CH, CH), c, :],
                    wbufs[j], gsem[j]).wait()
                pltpu.async_copy(wbufs[j], acc_s.at[sidx[j]], asem[j],
                                 add=True)

                @pl.when(i < RI - 1)
                def _(j=j, cc=cc):
                    pltpu.make_async_copy(wbufs[j], acc_s.at[sidx[j]],
                                          asem[j]).wait()
                    loadw(cc + RD, j)
            return carry
        lax.fori_loop(0, RI, scat, 0)
        for j in range(RD):
            pltpu.make_async_copy(wbufs[j], acc_s.at[sidx[j]],
                                  asem[j]).wait()
        offt = base + NF * CH
        pltpu.sync_copy(src_hbm.at[pl.ds(offt, TL)], idxt)
        pltpu.sync_copy(w_hbm.at[pl.ds(offt, TL), c, :], wbuft)
        pltpu.sync_copy(wbuft, acc_s.at[idxt], add=True)

        plsc.subcore_barrier()

        pltpu.sync_copy(acc_s.at[pl.ds(t * RT, RT), :],
                        out_hbm.at[pl.ds(t * RT, RT), c, :])

        @pl.when(t == NS - 1)
        def _():
            pltpu.sync_copy(acc_s.at[pl.ds(NS * RT, 16), :],
                            out_hbm.at[pl.ds(NS * RT, 16), c, :])

    return k(w2, src, zeros2d)


def kernel(x, edge_index, edge_attr, Wq, Wk, Wv, We, be):
    src = edge_index[0]
    dst = edge_index[1]
    qi, xi = _tc_query(x, Wq)
    xd, qs = _sc_gather(xi, qi, dst, src)
    v, eout, att_t = _tc_proj(edge_attr, xd, qs, Wk, Wv, We, be)
    coeff_flat = _sc_softmax(att_t.reshape(HH * EE), src)
    w2 = _tc_scale(coeff_flat.reshape(HH, EE).T, v)
    z2 = jnp.zeros((624, DD // 2), jnp.float32)
    out2 = _sc_scatter_out(w2, src, z2)
    return out2.reshape(NN, DD), eout
